# baseline (speedup 1.0000x reference)
import functools

import jax
import jax.numpy as jnp
from jax import lax
from jax.experimental import pallas as pl
from jax.experimental.pallas import tpu as pltpu
from jax.experimental.pallas import tpu_sc as plsc

_B, _S, _D = 4, 4096, 2048
_SURV = _S - (_S + 2) // 3
_NC, _NS = 2, 16
_CH = 24
_PARTS = (384, 768, 768, 810)


def _make_part(j0, nrows):
    nblocks, rem = divmod(nrows, 8)
    q, r2 = divmod(nblocks, 8)
    rows_q = 8 * q
    full = rows_q // 24
    assert rows_q % 24 == 0 and full >= 2 and r2 <= 7
    idxcap = -(-(rows_q + 8 + rem) // 16) * 16

    scratch = [
        pltpu.VMEM((idxcap,), jnp.int32),
        pltpu.VMEM((_CH, _D), jnp.float32),
        pltpu.VMEM((_CH, _D), jnp.float32),
        pltpu.VMEM((8, _D), jnp.float32),
        pltpu.VMEM((max(rem, 1), _D), jnp.float32),
        pltpu.SemaphoreType.DMA,
        pltpu.SemaphoreType.DMA,
        pltpu.SemaphoreType.DMA,
    ]

    @functools.partial(
        pl.kernel,
        mesh=plsc.VectorSubcoreMesh(core_axis_name="c", subcore_axis_name="s"),
        out_type=jax.ShapeDtypeStruct((_B, nrows, _D), jnp.float32),
        scratch_types=scratch,
    )
    def _part(table, out, idx_v, buf0, buf1, tb8, tbr, g0, g1, tsem):
        wid = lax.axis_index("s") * _NC + lax.axis_index("c")
        b = wid // 8
        o = wid % 8
        base = o * rows_q + 8 * jnp.minimum(o, r2)
        lanes = lax.iota(jnp.int32, 16)
        for i in range(idxcap // 16):
            j = j0 + base + i * 16 + lanes
            src = j + (j >> 1) + 1
            idx_v[pl.ds(i * 16, 16)] = jnp.minimum(src, _S - 1)

        plane = table.at[b]
        bufs = (buf0, buf1)
        gsems = (g0, g1)
        copies = [
            pltpu.async_copy(plane.at[idx_v.at[pl.ds(0, _CH)]], buf0, g0),
            pltpu.async_copy(plane.at[idx_v.at[pl.ds(_CH, _CH)]], buf1, g1),
        ]
        for t in range(full):
            s = t % 2
            copies[s].wait()
            pltpu.sync_copy(bufs[s], out.at[b, pl.ds(base + t * _CH, _CH)])
            nxt = t + 2
            if nxt < full:
                copies[s] = pltpu.async_copy(
                    plane.at[idx_v.at[pl.ds(nxt * _CH, _CH)]], bufs[s], gsems[s])

        if r2 > 0:
            @pl.when(o < r2)
            def _extra8():
                pltpu.async_copy(
                    plane.at[idx_v.at[pl.ds(rows_q, 8)]], tb8, tsem).wait()
                pltpu.sync_copy(tb8, out.at[b, pl.ds(base + rows_q, 8)])

        if rem > 0:
            @pl.when(o == 7)
            def _tail():
                pltpu.async_copy(
                    plane.at[idx_v.at[pl.ds(rows_q, rem)]], tbr, tsem).wait()
                pltpu.sync_copy(tbr, out.at[b, pl.ds(nrows - rem, rem)])

    return _part


_PART_FNS = []
_j0 = 0
for _R in _PARTS:
    _PART_FNS.append(_make_part(_j0, _R))
    _j0 += _R
assert _j0 == _SURV


def kernel(hidden_states):
    out = jnp.zeros((_B, _SURV, _D), jnp.float32)
    j0 = 0
    for fn, nrows in zip(_PART_FNS, _PARTS):
        out = lax.dynamic_update_slice(out, fn(hidden_states), (0, j0, 0))
        j0 += nrows
    return out


# speedup vs baseline: 2.6750x; 2.6750x over previous
import functools

import jax
import jax.numpy as jnp
from jax import lax
from jax.experimental import pallas as pl
from jax.experimental.pallas import tpu as pltpu
from jax.experimental.pallas import tpu_sc as plsc
from jax.experimental.layout import Layout, with_layout_constraint

_B, _S, _D = 4, 4096, 2048
_SURV = _S - (_S + 2) // 3
_NC, _NS = 2, 16
_TROWS = _B * _S * 16
_OROWS = _B * _SURV * 16
_FULL = 42


@functools.partial(
    pl.kernel,
    mesh=plsc.VectorSubcoreMesh(core_axis_name="c", subcore_axis_name="s"),
    out_type=jax.ShapeDtypeStruct((_OROWS, 128), jnp.float32),
    scratch_types=[
        pltpu.VMEM((43, 128), jnp.int32),
        pltpu.VMEM((128, 128), jnp.float32),
        pltpu.VMEM((128, 128), jnp.float32),
        pltpu.VMEM((64, 128), jnp.float32),
        pltpu.SemaphoreType.DMA,
        pltpu.SemaphoreType.DMA,
        pltpu.SemaphoreType.DMA,
    ],
)
def _prune(table, out, idx_v, buf0, buf1, tb, g0, g1, tsem):
    wid = lax.axis_index("s") * _NC + lax.axis_index("c")
    tok0 = wid * 85 + jnp.minimum(wid, 10)
    u0 = tok0 * 64
    lanes = lax.iota(jnp.int32, 16)

    def idx_body(t, carry):
        for k in range(8):
            u = (tok0 + 2 * t) * 64 + k * 16 + lanes
            j = u >> 6
            src = j + (j >> 1) + 1
            v = ((u & 3) * 65536 + (src >> 3) * 128
                 + ((u >> 2) & 15) * 8 + (src & 7))
            idx_v[t, pl.ds(k * 16, 16)] = jnp.minimum(v, _TROWS - 1)
        return carry

    lax.fori_loop(0, 43, idx_body, 0)

    bufs = (buf0, buf1)
    gsems = (g0, g1)
    copies = [
        pltpu.async_copy(table.at[idx_v.at[0]], buf0, g0),
        pltpu.async_copy(table.at[idx_v.at[1]], buf1, g1),
    ]
    for t in range(_FULL):
        s = t % 2
        copies[s].wait()
        pltpu.sync_copy(bufs[s], out.at[pl.ds(u0 + t * 128, 128)])
        nxt = t + 2
        if nxt < _FULL:
            copies[s] = pltpu.async_copy(
                table.at[idx_v.at[nxt]], bufs[s], gsems[s])

    @pl.when(wid < 10)
    def _last_full():
        pltpu.async_copy(table.at[idx_v.at[42]], buf0, tsem).wait()
        pltpu.sync_copy(buf0, out.at[pl.ds(u0 + _FULL * 128, 128)])

    @pl.when(wid >= 10)
    def _last_half():
        pltpu.async_copy(table.at[idx_v.at[42, pl.ds(0, 64)]], tb, tsem).wait()
        pltpu.sync_copy(tb, out.at[pl.ds(u0 + _FULL * 128, 64)])


def kernel(hidden_states):
    t5 = hidden_states.reshape(_B, _S // 8, 8, 16, 128)
    t5 = jnp.transpose(t5, (0, 1, 3, 2, 4))
    table = t5.reshape(_TROWS, 128)
    flat = _prune(table)
    v = flat.reshape(_SURV, 16, 4, 128)
    v = with_layout_constraint(
        v, Layout(major_to_minor=(0, 1, 2, 3), tiling=((4, 128),)))
    t = jnp.transpose(v, (2, 0, 1, 3))
    t = with_layout_constraint(
        t, Layout(major_to_minor=(1, 2, 0, 3), tiling=((4, 128),)))
    return t.reshape(_B, _SURV, _D)


# speedup vs baseline: 2.7742x; 1.0371x over previous
import functools

import jax
import jax.numpy as jnp
from jax import lax
from jax.experimental import pallas as pl
from jax.experimental.pallas import tpu as pltpu
from jax.experimental.pallas import tpu_sc as plsc
from jax.experimental.layout import Layout, with_layout_constraint

_B, _S, _D = 4, 4096, 2048
_SURV = _S - (_S + 2) // 3
_NC, _NS = 2, 16
_TROWS = _B * _S * 16
_OROWS = _B * _SURV * 16
_FULL = 42


@functools.partial(
    pl.kernel,
    mesh=plsc.VectorSubcoreMesh(core_axis_name="c", subcore_axis_name="s"),
    out_type=jax.ShapeDtypeStruct((_OROWS, 128), jnp.float32),
    scratch_types=[
        pltpu.VMEM((43, 128), jnp.int32),
        pltpu.VMEM((128, 128), jnp.float32),
        pltpu.VMEM((128, 128), jnp.float32),
        pltpu.VMEM((128, 128), jnp.float32),
        pltpu.VMEM((64, 128), jnp.float32),
        pltpu.SemaphoreType.DMA,
        pltpu.SemaphoreType.DMA,
        pltpu.SemaphoreType.DMA,
        pltpu.SemaphoreType.DMA,
        pltpu.SemaphoreType.DMA,
        pltpu.SemaphoreType.DMA,
        pltpu.SemaphoreType.DMA,
    ],
)
def _prune(table, out, idx_v, buf0, buf1, buf2, tb,
           g0, g1, g2, w0, w1, w2, tsem):
    wid = lax.axis_index("s") * _NC + lax.axis_index("c")
    tok0 = wid * 85 + jnp.minimum(wid, 10)
    u0 = tok0 * 64
    lanes = lax.iota(jnp.int32, 16)

    def gen_idx(t):
        for k in range(8):
            u = (tok0 + 2 * t) * 64 + k * 16 + lanes
            j = u >> 6
            src = j + (j >> 1) + 1
            v = ((u & 3) * 65536 + (src >> 3) * 128
                 + ((u >> 2) & 15) * 8 + (src & 7))
            idx_v[t, pl.ds(k * 16, 16)] = jnp.minimum(v, _TROWS - 1)

    bufs = (buf0, buf1, buf2)
    gsems = (g0, g1, g2)
    wsems = (w0, w1, w2)

    for t in range(3):
        gen_idx(t)
    copies = [
        pltpu.async_copy(table.at[idx_v.at[t]], bufs[t], gsems[t])
        for t in range(3)
    ]

    def idx_body(t, carry):
        gen_idx(t)
        return carry

    lax.fori_loop(3, 43, idx_body, 0, unroll=False)

    wcopies = [None, None, None]
    for t in range(_FULL):
        s = t % 3
        nxt = t + 2
        if t >= 1 and nxt < _FULL:
            sp = nxt % 3
            if wcopies[sp] is not None:
                wcopies[sp].wait()
            copies[sp] = pltpu.async_copy(
                table.at[idx_v.at[nxt]], bufs[sp], gsems[sp])
        copies[s].wait()
        wcopies[s] = pltpu.async_copy(
            bufs[s], out.at[pl.ds(u0 + t * 128, 128)], wsems[s])
    for s in range(3):
        if wcopies[s] is not None:
            wcopies[s].wait()

    @pl.when(wid < 10)
    def _last_full():
        pltpu.async_copy(table.at[idx_v.at[42]], buf0, tsem).wait()
        pltpu.sync_copy(buf0, out.at[pl.ds(u0 + _FULL * 128, 128)])

    @pl.when(wid >= 10)
    def _last_half():
        pltpu.async_copy(table.at[idx_v.at[42, pl.ds(0, 64)]], tb, tsem).wait()
        pltpu.sync_copy(tb, out.at[pl.ds(u0 + _FULL * 128, 64)])


def kernel(hidden_states):
    t5 = hidden_states.reshape(_B, _S // 8, 8, 16, 128)
    t5 = jnp.transpose(t5, (0, 1, 3, 2, 4))
    table = t5.reshape(_TROWS, 128)
    flat = _prune(table)
    v = flat.reshape(_SURV, 16, 4, 128)
    v = with_layout_constraint(
        v, Layout(major_to_minor=(0, 1, 2, 3), tiling=((4, 128),)))
    t = jnp.transpose(v, (2, 0, 1, 3))
    t = with_layout_constraint(
        t, Layout(major_to_minor=(1, 2, 0, 3), tiling=((4, 128),)))
    return t.reshape(_B, _SURV, _D)
